# Initial kernel scaffold; baseline (speedup 1.0000x reference)
import jax
import jax.numpy as jnp
from jax.experimental import pallas as pl


def kernel(node_features, edge_index, edge_features, W_node, b_node, W_edge, b_edge, Wm1, bm1, Wm2, bm2, Wu1, bu1, Wu2, bu2, ln_g, ln_b, W_r1, b_r1, W_r2, b_r2):
    raise NotImplementedError("write your pallas kernel here")


# speedup vs baseline: 2.9040x; 2.9040x over previous
import functools

import jax
import jax.numpy as jnp
from jax import lax
from jax.experimental import pallas as pl
from jax.experimental.pallas import tpu as pltpu
from jax.experimental.pallas import tpu_sc as plsc

N = 10000
E = 320000
NODE_DIM = 128
EDGE_DIM = 16
H = 128
HH = H // 2
L = 2
C = 10

N_PAD = 10240
NSUB = 16
CHUNK = 128
CPT = 160
EPT = CPT * CHUNK
E_PAD = NSUB * EPT
ROWS_PER_SUB = N_PAD // NSUB

_SC_PARAMS = pltpu.CompilerParams(use_tc_tiling_on_sc=False)


def _make_edge_kernel():
    out_type = [jax.ShapeDtypeStruct((2, N_PAD, HH), jnp.float32)]
    scratch = [
        pltpu.VMEM((CPT // 2, CHUNK), jnp.int32),
        pltpu.VMEM((CPT // 2, CHUNK), jnp.int32),
        pltpu.VMEM((2, CHUNK, HH), jnp.float32),
        pltpu.VMEM((2, CHUNK, HH), jnp.float32),
        pltpu.VMEM((2, CHUNK, HH), jnp.float32),
        pltpu.VMEM((CHUNK, HH), jnp.float32),
        pltpu.VMEM((CHUNK, HH), jnp.float32),
        pltpu.VMEM_SHARED((N_PAD, HH), jnp.float32),
        pltpu.SemaphoreType.DMA,
        pltpu.SemaphoreType.DMA,
    ]

    mesh = plsc.VectorSubcoreMesh(core_axis_name="c", subcore_axis_name="s")

    @functools.partial(pl.kernel, out_type=out_type, mesh=mesh,
                       compiler_params=_SC_PARAMS, scratch_types=scratch)
    def edge_kernel(ab_hbm, c_hbm, idx_hbm, s_out,
                    src_v, dst_v, a_v, b_v, c_v, h_v, z_v, s_sh,
                    gsem0, gsem1):
        cid = lax.axis_index("c")
        sid = lax.axis_index("s")

        def init_row(i, _):
            for k in range(HH // 16):
                z_v[i, pl.ds(k * 16, 16)] = jnp.zeros((16,), jnp.float32)
            return 0

        lax.fori_loop(0, CHUNK, init_row, 0)

        r0 = sid * ROWS_PER_SUB

        def zero_stripe(q, _):
            pltpu.sync_copy(z_v, s_sh.at[pl.ds(r0 + q * CHUNK, CHUNK)])
            return 0

        lax.fori_loop(0, ROWS_PER_SUB // CHUNK, zero_stripe, 0)
        plsc.subcore_barrier()

        a_src = ab_hbm.at[pl.ds(cid * N_PAD, N_PAD)]
        b_src = ab_hbm.at[pl.ds((2 + cid) * N_PAD, N_PAD)]

        cbase = cid * HH
        gsems = (gsem0, gsem1)
        HALF = CPT // 2

        for p in range(2):
            rows = pl.ds(p * HALF, HALF)
            pltpu.sync_copy(idx_hbm.at[sid].at[rows], src_v)
            pltpu.sync_copy(idx_hbm.at[NSUB + sid].at[rows], dst_v)
            ebase = sid * EPT + p * HALF * CHUNK

            def issue_gathers(j, slot, gsem):
                pltpu.make_async_copy(
                    a_src.at[src_v.at[j]], a_v.at[slot], gsem).start()
                pltpu.make_async_copy(
                    b_src.at[dst_v.at[j]], b_v.at[slot], gsem).start()
                pltpu.make_async_copy(
                    c_hbm.at[pl.ds(ebase + j * CHUNK, CHUNK),
                             pl.ds(cbase, HH)], c_v.at[slot], gsem).start()

            def wait_gathers(j, slot, gsem):
                pltpu.make_async_copy(
                    a_src.at[src_v.at[j]], a_v.at[slot], gsem).wait()
                pltpu.make_async_copy(
                    b_src.at[dst_v.at[j]], b_v.at[slot], gsem).wait()
                pltpu.make_async_copy(
                    c_hbm.at[pl.ds(ebase + j * CHUNK, CHUNK),
                             pl.ds(cbase, HH)], c_v.at[slot], gsem).wait()

            issue_gathers(0, 0, gsem0)
            issue_gathers(1, 1, gsem1)

            def chunk_pair(jj, _):
                for slot in range(2):
                    j = 2 * jj + slot
                    gsem = gsems[slot]
                    wait_gathers(j, slot, gsem)

                    def row_body(i, _):
                        for k in range(HH // 16):
                            sl = pl.ds(k * 16, 16)
                            h_v[i, sl] = jnp.maximum(
                                a_v[slot, i, sl] + b_v[slot, i, sl]
                                + c_v[slot, i, sl], 0.0)
                        return 0

                    lax.fori_loop(0, CHUNK, row_body, 0)

                    @pl.when(jj < HALF // 2 - 1)
                    def _():
                        issue_gathers(j + 2, slot, gsem)

                    pltpu.sync_copy(h_v, s_sh.at[dst_v.at[j]], add=True)
                return 0

            lax.fori_loop(0, HALF // 2, chunk_pair, 0)
        plsc.subcore_barrier()

        def dump(q, _):
            rows = pl.ds(r0 + q * CHUNK, CHUNK)
            pltpu.sync_copy(s_sh.at[rows], s_out.at[cid, rows])
            return 0

        lax.fori_loop(0, ROWS_PER_SUB // CHUNK, dump, 0)

    return edge_kernel


def _make_deg_kernel():
    mesh = plsc.VectorSubcoreMesh(core_axis_name="c", subcore_axis_name="s")

    @functools.partial(
        pl.kernel,
        out_type=[jax.ShapeDtypeStruct((2, N_PAD, 16), jnp.float32)],
        mesh=mesh, compiler_params=_SC_PARAMS,
        scratch_types=[
            pltpu.VMEM((CPT, CHUNK), jnp.int32),
            pltpu.VMEM((CHUNK, 16), jnp.float32),
            pltpu.VMEM((CHUNK, 16), jnp.float32),
            pltpu.VMEM_SHARED((N_PAD, 16), jnp.float32),
            pltpu.SemaphoreType.DMA,
        ],
    )
    def deg_kernel(idx_hbm, deg_out, dst_v, z16_v, ones_v, deg_sh, sem):
        cid = lax.axis_index("c")
        sid = lax.axis_index("s")

        def init_row(i, _):
            z16_v[i, pl.ds(0, 16)] = jnp.zeros((16,), jnp.float32)
            ones_v[i, pl.ds(0, 16)] = jnp.ones((16,), jnp.float32)
            return 0

        lax.fori_loop(0, CHUNK, init_row, 0)
        r0 = sid * ROWS_PER_SUB

        def zero_stripe(q, _):
            pltpu.sync_copy(z16_v, deg_sh.at[pl.ds(r0 + q * CHUNK, CHUNK)])
            return 0

        lax.fori_loop(0, ROWS_PER_SUB // CHUNK, zero_stripe, 0)
        plsc.subcore_barrier()

        pltpu.sync_copy(idx_hbm.at[NSUB + sid], dst_v)
        jlo = cid * (CPT // 2)

        def chunk_body(j, _):
            pltpu.sync_copy(ones_v, deg_sh.at[dst_v.at[jlo + j]], add=True)
            return 0

        lax.fori_loop(0, CPT // 2, chunk_body, 0)
        plsc.subcore_barrier()

        def dump(q, _):
            rows = pl.ds(r0 + q * CHUNK, CHUNK)
            pltpu.sync_copy(deg_sh.at[rows], deg_out.at[cid, rows])
            return 0

        lax.fori_loop(0, ROWS_PER_SUB // CHUNK, dump, 0)

    return deg_kernel


_edge = _make_edge_kernel()
_deg = _make_deg_kernel()


_BLK = 512
_GRID = N_PAD // _BLK

_row_spec = pl.BlockSpec((_BLK, H), lambda i: (i, 0))
_half_spec = pl.BlockSpec((2, _BLK, HH), lambda i: (0, i, 0))
_w_spec = pl.BlockSpec((H, H), lambda i: (0, 0))
_b_spec = pl.BlockSpec((1, H), lambda i: (0, 0))


def _split_dots(x, w, bias=None):
    left = jnp.dot(x, w[:, :HH], preferred_element_type=jnp.float32)
    right = jnp.dot(x, w[:, HH:], preferred_element_type=jnp.float32)
    if bias is not None:
        left = left + bias[:, :HH]
        right = right + bias[:, HH:]
    return jnp.stack([left, right])


def _embed_ab_body(nf, wn, bn, w1a, w1b, bm1, x_o, a_o, b_o):
    x = jnp.dot(nf[...], wn[...], preferred_element_type=jnp.float32) + bn[...]
    x_o[...] = x
    a_o[...] = _split_dots(x, w1a[...])
    b_o[...] = _split_dots(x, w1b[...], bm1[...])


def _ab_body(x, w1a, w1b, bm1, a_o, b_o):
    xv = x[...]
    a_o[...] = _split_dots(xv, w1a[...])
    b_o[...] = _split_dots(xv, w1b[...], bm1[...])


_ab_out = [jax.ShapeDtypeStruct((2, N_PAD, HH), jnp.float32)] * 2


def _embed_ab(nf_p, wn, bn, w1a, w1b, bm1):
    return pl.pallas_call(
        _embed_ab_body,
        grid=(_GRID,),
        in_specs=[_row_spec, _w_spec, _b_spec, _w_spec, _w_spec, _b_spec],
        out_specs=[_row_spec, _half_spec, _half_spec],
        out_shape=[jax.ShapeDtypeStruct((N_PAD, H), jnp.float32)] + _ab_out,
    )(nf_p, wn, bn, w1a, w1b, bm1)


def _ab(x, w1a, w1b, bm1):
    return pl.pallas_call(
        _ab_body,
        grid=(_GRID,),
        in_specs=[_row_spec, _w_spec, _w_spec, _b_spec],
        out_specs=[_half_spec, _half_spec],
        out_shape=list(_ab_out),
    )(x, w1a, w1b, bm1)


_EBLK = 1024
_EGRID = E_PAD // _EBLK


def _cmsg_body(ef, wc, bc, c_o):
    c_o[...] = jnp.dot(ef[...], wc[...],
                       preferred_element_type=jnp.float32) + bc[...]


def _cmsg(ef_p, wc, bc):
    return pl.pallas_call(
        _cmsg_body,
        grid=(_EGRID,),
        in_specs=[
            pl.BlockSpec((_EBLK, EDGE_DIM), lambda i: (i, 0)),
            pl.BlockSpec((EDGE_DIM, H), lambda i: (0, 0)),
            _b_spec,
        ],
        out_specs=pl.BlockSpec((_EBLK, H), lambda i: (i, 0)),
        out_shape=jax.ShapeDtypeStruct((E_PAD, H), jnp.float32),
    )(ef_p, wc, bc)


def _update_body(s0, s1, d0, d1, x, wm2a, wm2b, bm2, wu1a, wu1b, bu1, wu2,
                 bu2, g, b, xo):
    deg = d0[...][:, 0:1] + d1[...][:, 0:1]
    agg = (jnp.dot(s0[...], wm2a[...], preferred_element_type=jnp.float32)
           + jnp.dot(s1[...], wm2b[...], preferred_element_type=jnp.float32)
           + deg * bm2[...])
    xv = x[...]
    h2 = jnp.maximum(
        jnp.dot(xv, wu1a[...], preferred_element_type=jnp.float32)
        + jnp.dot(agg, wu1b[...], preferred_element_type=jnp.float32)
        + bu1[...], 0.0)
    upd = jnp.dot(h2, wu2[...], preferred_element_type=jnp.float32) + bu2[...]
    y = xv + upd
    mu = jnp.mean(y, axis=1, keepdims=True)
    yc = y - mu
    var = jnp.mean(yc * yc, axis=1, keepdims=True)
    xo[...] = yc * lax.rsqrt(var + 1e-5) * g[...] + b[...]


def _update(s_pair, deg_pair, x, wm2, bm2, wu1a, wu1b, bu1, wu2, bu2, g, b):
    hspec = pl.BlockSpec((_BLK, HH), lambda i: (i, 0))
    dspec = pl.BlockSpec((_BLK, 16), lambda i: (i, 0))
    hw_spec = pl.BlockSpec((HH, H), lambda i: (0, 0))
    return pl.pallas_call(
        _update_body,
        grid=(_GRID,),
        in_specs=[hspec, hspec, dspec, dspec, _row_spec,
                  hw_spec, hw_spec, _b_spec, _w_spec, _w_spec, _b_spec,
                  _w_spec, _b_spec, _b_spec, _b_spec],
        out_specs=_row_spec,
        out_shape=jax.ShapeDtypeStruct((N_PAD, H), jnp.float32),
    )(s_pair[0], s_pair[1], deg_pair[0], deg_pair[1], x,
      wm2[:HH], wm2[HH:], bm2, wu1a, wu1b, bu1, wu2, bu2, g, b)


_RBLK = 2000
_RGRID = N // _RBLK


def _readout_body(x, wr1, br1, wr2, br2, out, acc):
    i = pl.program_id(0)
    part = jnp.sum(x[...], axis=0, keepdims=True)

    @pl.when(i == 0)
    def _():
        acc[...] = part

    @pl.when(i > 0)
    def _():
        acc[...] += part

    @pl.when(i == _RGRID - 1)
    def _():
        gm = acc[...] * (1.0 / N)
        h3 = jnp.maximum(
            jnp.dot(gm, wr1[...], preferred_element_type=jnp.float32)
            + br1[...], 0.0)
        out[...] = jnp.dot(h3, wr2[...],
                           preferred_element_type=jnp.float32) + br2[...]


def _readout(x, wr1, br1, wr2p, br2p):
    return pl.pallas_call(
        _readout_body,
        grid=(_RGRID,),
        in_specs=[
            pl.BlockSpec((_RBLK, H), lambda i: (i, 0)),
            _w_spec, _b_spec, _w_spec, _b_spec,
        ],
        out_specs=pl.BlockSpec((1, H), lambda i: (0, 0)),
        out_shape=jax.ShapeDtypeStruct((1, H), jnp.float32),
        scratch_shapes=[pltpu.VMEM((1, H), jnp.float32)],
    )(x, wr1, br1, wr2p, br2p)


def kernel(node_features, edge_index, edge_features, W_node, b_node, W_edge,
           b_edge, Wm1, bm1, Wm2, bm2, Wu1, bu1, Wu2, bu2, ln_g, ln_b,
           W_r1, b_r1, W_r2, b_r2):
    f32 = jnp.float32
    pad_e = E_PAD - E
    idx3 = jnp.concatenate([
        edge_index[0], jnp.zeros((pad_e,), jnp.int32),
        edge_index[1], jnp.full((pad_e,), N_PAD - 1, jnp.int32),
    ]).reshape(2 * NSUB, CPT, CHUNK)
    ef_p = jnp.pad(edge_features, ((0, pad_e), (0, 0)))
    nf_p = jnp.pad(node_features, ((0, N_PAD - N), (0, 0)))

    def rowvec(v):
        return v.reshape(1, -1).astype(f32)

    x = None
    (deg_pair,) = _deg(idx3)
    for l in range(L):
        w1a = Wm1[l][:H]
        w1b = Wm1[l][H:2 * H]
        w1c = Wm1[l][2 * H:]
        wc = W_edge @ w1c
        bc = rowvec(b_edge @ w1c)
        if l == 0:
            x, a2, b2 = _embed_ab(nf_p, W_node, rowvec(b_node), w1a, w1b,
                                  rowvec(bm1[l]))
        else:
            a2, b2 = _ab(x, w1a, w1b, rowvec(bm1[l]))
        ab = jnp.concatenate(
            [a2.reshape(2 * N_PAD, HH), b2.reshape(2 * N_PAD, HH)])
        c_e = _cmsg(ef_p, wc, bc)
        (s_pair,) = _edge(ab, c_e, idx3)
        x = _update(s_pair, deg_pair, x, Wm2[l], rowvec(bm2[l]),
                    Wu1[l][:H], Wu1[l][H:], rowvec(bu1[l]), Wu2[l],
                    rowvec(bu2[l]), rowvec(ln_g[l]), rowvec(ln_b[l]))

    wr2p = jnp.pad(W_r2, ((0, 0), (0, H - C)))
    br2p = jnp.pad(b_r2, (0, H - C)).reshape(1, H)
    out = _readout(x, W_r1, rowvec(b_r1), wr2p, br2p)
    return out[:, :C]


# speedup vs baseline: 2.9068x; 1.0009x over previous
import functools

import jax
import jax.numpy as jnp
from jax import lax
from jax.experimental import pallas as pl
from jax.experimental.pallas import tpu as pltpu
from jax.experimental.pallas import tpu_sc as plsc

N = 10000
E = 320000
NODE_DIM = 128
EDGE_DIM = 16
H = 128
HH = H // 2
L = 2
C = 10

N_PAD = 10240
NSUB = 16
CHUNK = 128
CPT = 160
EPT = CPT * CHUNK
E_PAD = NSUB * EPT
ROWS_PER_SUB = N_PAD // NSUB

_SC_PARAMS = pltpu.CompilerParams(use_tc_tiling_on_sc=False)


def _make_edge_kernel():
    out_type = [jax.ShapeDtypeStruct((2, N_PAD, HH), jnp.float32)]
    scratch = [
        pltpu.VMEM((CPT // 2, CHUNK), jnp.int32),
        pltpu.VMEM((CPT // 2, CHUNK), jnp.int32),
        pltpu.VMEM((2, CHUNK, HH), jnp.float32),
        pltpu.VMEM((2, CHUNK, HH), jnp.float32),
        pltpu.VMEM((2, CHUNK, HH), jnp.float32),
        pltpu.VMEM((CHUNK, HH), jnp.float32),
        pltpu.VMEM((CHUNK, HH), jnp.float32),
        pltpu.VMEM_SHARED((N_PAD, HH), jnp.float32),
        pltpu.SemaphoreType.DMA,
        pltpu.SemaphoreType.DMA,
    ]

    mesh = plsc.VectorSubcoreMesh(core_axis_name="c", subcore_axis_name="s")

    @functools.partial(pl.kernel, out_type=out_type, mesh=mesh,
                       compiler_params=_SC_PARAMS, scratch_types=scratch)
    def edge_kernel(ab_hbm, c_hbm, idx_hbm, s_out,
                    src_v, dst_v, a_v, b_v, c_v, h_v, z_v, s_sh,
                    gsem0, gsem1):
        cid = lax.axis_index("c")
        sid = lax.axis_index("s")

        def init_row(i, _):
            for k in range(HH // 16):
                z_v[i, pl.ds(k * 16, 16)] = jnp.zeros((16,), jnp.float32)
            return 0

        lax.fori_loop(0, CHUNK, init_row, 0)

        r0 = sid * ROWS_PER_SUB

        def zero_stripe(q, _):
            pltpu.sync_copy(z_v, s_sh.at[pl.ds(r0 + q * CHUNK, CHUNK)])
            return 0

        lax.fori_loop(0, ROWS_PER_SUB // CHUNK, zero_stripe, 0)
        plsc.subcore_barrier()

        a_src = ab_hbm.at[pl.ds(cid * N_PAD, N_PAD)]
        b_src = ab_hbm.at[pl.ds((2 + cid) * N_PAD, N_PAD)]

        cbase = cid * HH
        gsems = (gsem0, gsem1)
        HALF = CPT // 2

        for p in range(2):
            rows = pl.ds(p * HALF, HALF)
            pltpu.sync_copy(idx_hbm.at[sid].at[rows], src_v)
            pltpu.sync_copy(idx_hbm.at[NSUB + sid].at[rows], dst_v)
            ebase = sid * EPT + p * HALF * CHUNK

            def issue_gathers(j, slot, gsem):
                pltpu.make_async_copy(
                    a_src.at[src_v.at[j]], a_v.at[slot], gsem).start()
                pltpu.make_async_copy(
                    b_src.at[dst_v.at[j]], b_v.at[slot], gsem).start()
                pltpu.make_async_copy(
                    c_hbm.at[pl.ds(ebase + j * CHUNK, CHUNK),
                             pl.ds(cbase, HH)], c_v.at[slot], gsem).start()

            def wait_gathers(j, slot, gsem):
                pltpu.make_async_copy(
                    a_src.at[src_v.at[j]], a_v.at[slot], gsem).wait()
                pltpu.make_async_copy(
                    b_src.at[dst_v.at[j]], b_v.at[slot], gsem).wait()
                pltpu.make_async_copy(
                    c_hbm.at[pl.ds(ebase + j * CHUNK, CHUNK),
                             pl.ds(cbase, HH)], c_v.at[slot], gsem).wait()

            issue_gathers(0, 0, gsem0)
            issue_gathers(1, 1, gsem1)

            def chunk_pair(jj, _):
                for slot in range(2):
                    j = 2 * jj + slot
                    gsem = gsems[slot]
                    wait_gathers(j, slot, gsem)

                    def row_body(i, _):
                        for k in range(HH // 16):
                            sl = pl.ds(k * 16, 16)
                            h_v[i, sl] = jnp.maximum(
                                a_v[slot, i, sl] + b_v[slot, i, sl]
                                + c_v[slot, i, sl], 0.0)
                        return 0

                    lax.fori_loop(0, CHUNK, row_body, 0)

                    @pl.when(jj < HALF // 2 - 1)
                    def _():
                        issue_gathers(j + 2, slot, gsem)

                    pltpu.sync_copy(h_v, s_sh.at[dst_v.at[j]], add=True)
                return 0

            lax.fori_loop(0, HALF // 2, chunk_pair, 0)
        plsc.subcore_barrier()

        def dump(q, _):
            rows = pl.ds(r0 + q * CHUNK, CHUNK)
            pltpu.sync_copy(s_sh.at[rows], s_out.at[cid, rows])
            return 0

        lax.fori_loop(0, ROWS_PER_SUB // CHUNK, dump, 0)

    return edge_kernel


def _make_deg_kernel():
    mesh = plsc.VectorSubcoreMesh(core_axis_name="c", subcore_axis_name="s")

    @functools.partial(
        pl.kernel,
        out_type=[jax.ShapeDtypeStruct((2, N_PAD, 16), jnp.float32)],
        mesh=mesh, compiler_params=_SC_PARAMS,
        scratch_types=[
            pltpu.VMEM((CPT, CHUNK), jnp.int32),
            pltpu.VMEM((CHUNK, 16), jnp.float32),
            pltpu.VMEM((CHUNK, 16), jnp.float32),
            pltpu.VMEM_SHARED((N_PAD, 16), jnp.float32),
            pltpu.SemaphoreType.DMA,
        ],
    )
    def deg_kernel(idx_hbm, deg_out, dst_v, z16_v, ones_v, deg_sh, sem):
        cid = lax.axis_index("c")
        sid = lax.axis_index("s")

        def init_row(i, _):
            z16_v[i, pl.ds(0, 16)] = jnp.zeros((16,), jnp.float32)
            ones_v[i, pl.ds(0, 16)] = jnp.ones((16,), jnp.float32)
            return 0

        lax.fori_loop(0, CHUNK, init_row, 0)
        r0 = sid * ROWS_PER_SUB

        def zero_stripe(q, _):
            pltpu.sync_copy(z16_v, deg_sh.at[pl.ds(r0 + q * CHUNK, CHUNK)])
            return 0

        lax.fori_loop(0, ROWS_PER_SUB // CHUNK, zero_stripe, 0)
        plsc.subcore_barrier()

        pltpu.sync_copy(idx_hbm.at[NSUB + sid], dst_v)
        jlo = cid * (CPT // 2)

        def chunk_body(j, _):
            pltpu.sync_copy(ones_v, deg_sh.at[dst_v.at[jlo + j]], add=True)
            return 0

        lax.fori_loop(0, CPT // 2, chunk_body, 0)
        plsc.subcore_barrier()

        def dump(q, _):
            rows = pl.ds(r0 + q * CHUNK, CHUNK)
            pltpu.sync_copy(deg_sh.at[rows], deg_out.at[cid, rows])
            return 0

        lax.fori_loop(0, ROWS_PER_SUB // CHUNK, dump, 0)

    return deg_kernel


_edge = _make_edge_kernel()
_deg = _make_deg_kernel()


_BLK = 512
_GRID = N_PAD // _BLK

_row_spec = pl.BlockSpec((_BLK, H), lambda i: (i, 0))
_half_spec = pl.BlockSpec((2, _BLK, HH), lambda i: (0, i, 0))
_w_spec = pl.BlockSpec((H, H), lambda i: (0, 0))
_b_spec = pl.BlockSpec((1, H), lambda i: (0, 0))


def _split_dots(x, w, bias=None):
    left = jnp.dot(x, w[:, :HH], preferred_element_type=jnp.float32)
    right = jnp.dot(x, w[:, HH:], preferred_element_type=jnp.float32)
    if bias is not None:
        left = left + bias[:, :HH]
        right = right + bias[:, HH:]
    return jnp.stack([left, right])


def _embed_ab_body(nf, wn, bn, w1a, w1b, bm1, x_o, a_o, b_o):
    x = jnp.dot(nf[...], wn[...], preferred_element_type=jnp.float32) + bn[...]
    x_o[...] = x
    a_o[...] = _split_dots(x, w1a[...])
    b_o[...] = _split_dots(x, w1b[...], bm1[...])


def _ab_body(x, w1a, w1b, bm1, a_o, b_o):
    xv = x[...]
    a_o[...] = _split_dots(xv, w1a[...])
    b_o[...] = _split_dots(xv, w1b[...], bm1[...])


_ab_out = [jax.ShapeDtypeStruct((2, N_PAD, HH), jnp.float32)] * 2


def _embed_ab(nf_p, wn, bn, w1a, w1b, bm1):
    return pl.pallas_call(
        _embed_ab_body,
        grid=(_GRID,),
        in_specs=[_row_spec, _w_spec, _b_spec, _w_spec, _w_spec, _b_spec],
        out_specs=[_row_spec, _half_spec, _half_spec],
        out_shape=[jax.ShapeDtypeStruct((N_PAD, H), jnp.float32)] + _ab_out,
    )(nf_p, wn, bn, w1a, w1b, bm1)


def _ab(x, w1a, w1b, bm1):
    return pl.pallas_call(
        _ab_body,
        grid=(_GRID,),
        in_specs=[_row_spec, _w_spec, _w_spec, _b_spec],
        out_specs=[_half_spec, _half_spec],
        out_shape=list(_ab_out),
    )(x, w1a, w1b, bm1)


_EBLK = 1024
_EGRID = E_PAD // _EBLK


def _cmsg_body(ef, wc, bc, c_o):
    c_o[...] = jnp.dot(ef[...], wc[...],
                       preferred_element_type=jnp.float32) + bc[...]


def _cmsg(ef_p, wc, bc):
    return pl.pallas_call(
        _cmsg_body,
        grid=(_EGRID,),
        in_specs=[
            pl.BlockSpec((_EBLK, EDGE_DIM), lambda i: (i, 0)),
            pl.BlockSpec((EDGE_DIM, H), lambda i: (0, 0)),
            _b_spec,
        ],
        out_specs=pl.BlockSpec((_EBLK, H), lambda i: (i, 0)),
        out_shape=jax.ShapeDtypeStruct((E_PAD, H), jnp.float32),
    )(ef_p, wc, bc)


def _update_body(s0, s1, d0, d1, x, wm2a, wm2b, bm2, wu1a, wu1b, bu1, wu2,
                 bu2, g, b, xo):
    deg = d0[...][:, 0:1] + d1[...][:, 0:1]
    agg = (jnp.dot(s0[...], wm2a[...], preferred_element_type=jnp.float32)
           + jnp.dot(s1[...], wm2b[...], preferred_element_type=jnp.float32)
           + deg * bm2[...])
    xv = x[...]
    h2 = jnp.maximum(
        jnp.dot(xv, wu1a[...], preferred_element_type=jnp.float32)
        + jnp.dot(agg, wu1b[...], preferred_element_type=jnp.float32)
        + bu1[...], 0.0)
    upd = jnp.dot(h2, wu2[...], preferred_element_type=jnp.float32) + bu2[...]
    y = xv + upd
    mu = jnp.mean(y, axis=1, keepdims=True)
    yc = y - mu
    var = jnp.mean(yc * yc, axis=1, keepdims=True)
    xo[...] = yc * lax.rsqrt(var + 1e-5) * g[...] + b[...]


def _update(s_pair, deg_pair, x, wm2, bm2, wu1a, wu1b, bu1, wu2, bu2, g, b):
    hspec = pl.BlockSpec((_BLK, HH), lambda i: (i, 0))
    dspec = pl.BlockSpec((_BLK, 16), lambda i: (i, 0))
    hw_spec = pl.BlockSpec((HH, H), lambda i: (0, 0))
    return pl.pallas_call(
        _update_body,
        grid=(_GRID,),
        in_specs=[hspec, hspec, dspec, dspec, _row_spec,
                  hw_spec, hw_spec, _b_spec, _w_spec, _w_spec, _b_spec,
                  _w_spec, _b_spec, _b_spec, _b_spec],
        out_specs=_row_spec,
        out_shape=jax.ShapeDtypeStruct((N_PAD, H), jnp.float32),
    )(s_pair[0], s_pair[1], deg_pair[0], deg_pair[1], x,
      wm2[:HH], wm2[HH:], bm2, wu1a, wu1b, bu1, wu2, bu2, g, b)


_RBLK = 2000
_RGRID = N // _RBLK


def _readout_body(x, wr1, br1, wr2, br2, out, acc):
    i = pl.program_id(0)
    part = jnp.sum(x[...], axis=0, keepdims=True)

    @pl.when(i == 0)
    def _():
        acc[...] = part

    @pl.when(i > 0)
    def _():
        acc[...] += part

    @pl.when(i == _RGRID - 1)
    def _():
        gm = acc[...] * (1.0 / N)
        h3 = jnp.maximum(
            jnp.dot(gm, wr1[...], preferred_element_type=jnp.float32)
            + br1[...], 0.0)
        out[...] = jnp.dot(h3, wr2[...],
                           preferred_element_type=jnp.float32) + br2[...]


def _readout(x, wr1, br1, wr2p, br2p):
    return pl.pallas_call(
        _readout_body,
        grid=(_RGRID,),
        in_specs=[
            pl.BlockSpec((_RBLK, H), lambda i: (i, 0)),
            _w_spec, _b_spec, _w_spec, _b_spec,
        ],
        out_specs=pl.BlockSpec((1, H), lambda i: (0, 0)),
        out_shape=jax.ShapeDtypeStruct((1, H), jnp.float32),
        scratch_shapes=[pltpu.VMEM((1, H), jnp.float32)],
    )(x, wr1, br1, wr2p, br2p)


def kernel(node_features, edge_index, edge_features, W_node, b_node, W_edge,
           b_edge, Wm1, bm1, Wm2, bm2, Wu1, bu1, Wu2, bu2, ln_g, ln_b,
           W_r1, b_r1, W_r2, b_r2):
    f32 = jnp.float32
    pad_e = E_PAD - E
    idx3 = jnp.concatenate([
        edge_index[0], jnp.zeros((pad_e,), jnp.int32),
        edge_index[1], jnp.full((pad_e,), N_PAD - 1, jnp.int32),
    ]).reshape(2 * NSUB, CPT, CHUNK)
    ef_p = jnp.pad(edge_features, ((0, pad_e), (0, 0)))
    nf_p = jnp.pad(node_features, ((0, N_PAD - N), (0, 0)))

    def rowvec(v):
        return v.reshape(1, -1).astype(f32)

    x = None
    (deg_pair,) = _deg(idx3)
    c_all = []
    for l in range(L):
        w1c = Wm1[l][2 * H:]
        wc = W_edge @ w1c
        bc = rowvec(b_edge @ w1c)
        c_all.append(_cmsg(ef_p, wc, bc))
    for l in range(L):
        w1a = Wm1[l][:H]
        w1b = Wm1[l][H:2 * H]
        if l == 0:
            x, a2, b2 = _embed_ab(nf_p, W_node, rowvec(b_node), w1a, w1b,
                                  rowvec(bm1[l]))
        else:
            a2, b2 = _ab(x, w1a, w1b, rowvec(bm1[l]))
        ab = jnp.concatenate(
            [a2.reshape(2 * N_PAD, HH), b2.reshape(2 * N_PAD, HH)])
        (s_pair,) = _edge(ab, c_all[l], idx3)
        x = _update(s_pair, deg_pair, x, Wm2[l], rowvec(bm2[l]),
                    Wu1[l][:H], Wu1[l][H:], rowvec(bu1[l]), Wu2[l],
                    rowvec(bu2[l]), rowvec(ln_g[l]), rowvec(ln_b[l]))

    wr2p = jnp.pad(W_r2, ((0, 0), (0, H - C)))
    br2p = jnp.pad(b_r2, (0, H - C)).reshape(1, H)
    out = _readout(x, W_r1, rowvec(b_r1), wr2p, br2p)
    return out[:, :C]
